# baseline (speedup 1.0000x reference)
import jax, jax.numpy as jnp
from jax.experimental import pallas as pl

def _copy_body(x_ref, o_ref):
    o_ref[...] = x_ref[...]

def kernel(x, edge_index, edge_attr, batch, action, params):
    row, col = edge_index[0], edge_index[1]
    h = jnp.concatenate([x, action], axis=-1)
    h = jax.nn.relu(h @ params["Wx"] + params["bx"])
    h = pl.pallas_call(_copy_body, out_shape=jax.ShapeDtypeStruct(h.shape, h.dtype))(h)
    ea = jax.nn.relu(edge_attr @ params["We"] + params["be"])
    def _segment_mean(data, ids, num_segments):
        s = jax.ops.segment_sum(data, ids, num_segments=num_segments)
        c = jax.ops.segment_sum(jnp.ones((data.shape[0], 1), data.dtype), ids, num_segments=num_segments)
        return s / jnp.clip(c, 1.0, None)
    def _layernorm(x, g, b):
        m = jnp.mean(x, axis=-1, keepdims=True)
        v = jnp.mean((x - m) ** 2, axis=-1, keepdims=True)
        return (x - m) / jnp.sqrt(v + 1e-5) * g + b
    for blk in params["blocks"]:
        e_in = jnp.concatenate([h[row], h[col], ea], axis=-1)
        e2 = ea + (jax.nn.relu(e_in @ blk["We1"] + blk["be1"]) @ blk["We2"] + blk["be2"])
        agg = _segment_mean(e2, col, 10000)
        n_in = jnp.concatenate([h, agg], axis=-1)
        x2 = h + (jax.nn.relu(n_in @ blk["Wn1"] + blk["bn1"]) @ blk["Wn2"] + blk["bn2"])
        h = _layernorm(x2, blk["gx"], blk["bxn"])
        ea = _layernorm(e2, blk["ge"], blk["ben"])
    v = h @ params["Wout"] + params["bout"]
    return _segment_mean(v, batch, 64)


# speedup vs baseline: 1.9468x; 1.9468x over previous
import functools

import jax
import jax.numpy as jnp
from jax import lax
from jax.experimental import pallas as pl
from jax.experimental.pallas import tpu as pltpu
from jax.experimental.pallas import tpu_sc as plsc

N = 10000
E = 320000
NODE_D = 64
EDGE_D = 32
NUM_GRAPHS = 64

NC = 2
NS = 16
NW = NC * NS
EPW = E // NW
CHUNK = 80
NCHUNK = EPW // CHUNK

EG = 4
NG = 4
EPG = E // (NC * EG)
SCH = 200
SNCH = EPG // SCH
QS = 2560

TE = 8000
GRID_E = E // TE


def _mesh():
    return plsc.VectorSubcoreMesh(core_axis_name="c", subcore_axis_name="s")


@functools.partial(
    pl.kernel,
    out_type=jax.ShapeDtypeStruct((E, NODE_D), jnp.float32),
    mesh=_mesh(),
    scratch_types=[
        pltpu.VMEM((CHUNK,), jnp.int32),
        pltpu.VMEM((CHUNK,), jnp.int32),
        pltpu.VMEM((CHUNK, 2 * NODE_D), jnp.float32),
        pltpu.VMEM((CHUNK, 2 * NODE_D), jnp.float32),
        pltpu.VMEM((CHUNK, NODE_D), jnp.float32),
        pltpu.SemaphoreType.DMA,
    ],
)
def _gather_sum(t_hbm, row_hbm, col_hbm, out_hbm,
                idx_r, idx_c, bufr, bufc, bufs, sem):
    wid = lax.axis_index("s") * NC + lax.axis_index("c")
    base0 = wid * EPW

    def chunk_body(j, carry):
        base = base0 + j * CHUNK
        pltpu.sync_copy(row_hbm.at[pl.ds(base, CHUNK)], idx_r)
        pltpu.sync_copy(col_hbm.at[pl.ds(base, CHUNK)], idx_c)
        d1 = pltpu.async_copy(t_hbm.at[idx_r], bufr, sem)
        d2 = pltpu.async_copy(t_hbm.at[idx_c], bufc, sem)
        d1.wait()
        d2.wait()

        def add_row(i, c2):
            for k in range(NODE_D // 16):
                sl = pl.ds(k * 16, 16)
                sl_q = pl.ds(NODE_D + k * 16, 16)
                bufs[i, sl] = bufr[i, sl] + bufc[i, sl_q]
            return c2

        lax.fori_loop(0, CHUNK, add_row, 0)
        pltpu.sync_copy(bufs, out_hbm.at[pl.ds(base, CHUNK)])
        return carry

    lax.fori_loop(0, NCHUNK, chunk_body, 0)


def _scatter_like(width, values_fn):
    pack = 128 // width
    qp = QS // pack
    qp_last = (N - (NG - 1) * QS) // pack
    sub = 40
    nred = qp // sub
    dummy_row = qp
    tmp0 = qp + 8
    inc0 = tmp0 + sub
    acc_r = inc0 + sub
    shift = pack.bit_length() - 1
    npacked = N // pack

    full_last = qp_last // sub
    rem = -(-(qp_last - full_last * sub) // 8) * 8
    np_out = (NG - 1) * qp + full_last * sub + rem

    def body(refs, e2_v, idx_v, acc, shared):
        c = lax.axis_index("c")
        s = lax.axis_index("s")
        eg = s // NG
        ng = s % NG
        lo = ng * QS
        hi = jnp.minimum(lo + QS, N)
        zero16 = jnp.zeros((16,), jnp.float32)
        iota16 = lax.iota(jnp.int32, 16)
        zero_i16 = iota16 * 0
        one_lane = (1 - jnp.minimum(iota16, 1)).astype(jnp.float32)
        e2_hbm, col_hbm, out_hbm = refs

        def fill_zero(i, carry):
            for k in range(8):
                acc[i, pl.ds(k * 16, 16)] = zero16
            return carry

        lax.fori_loop(0, acc_r, fill_zero, 0)

        base_e = (c * EG + eg) * EPG

        def chunk_body(j, carry):
            base = base_e + j * SCH
            pltpu.sync_copy(col_hbm.at[pl.ds(base, SCH)], idx_v)
            if e2_hbm is not None:
                pltpu.sync_copy(e2_hbm.at[pl.ds(base, SCH)], e2_v)

            def group_body(g, c2):
                cids = idx_v[pl.ds(g * 16, 16)]
                q = cids - lo
                ok = (cids >= lo) & (cids < hi)
                rows = jnp.where(ok, lax.shift_right_logical(q, shift),
                                 dummy_row)
                offs = jnp.where(ok, (q & (pack - 1)) * width, 0)
                for l in range(16):
                    i = g * 16 + l
                    r16 = zero_i16 + rows[l]
                    for val, lidx in values_fn(e2_v, i, iota16, one_lane):
                        plsc.addupdate_scatter(acc, [r16, lidx + offs[l]],
                                               val)
                return c2

            lax.fori_loop(0, SCH // 16, group_body, 0)
            return carry

        lax.fori_loop(0, SNCH, chunk_body, 0)

        def add_chunk(i, c2):
            for k in range(8):
                sl = pl.ds(k * 16, 16)
                acc[inc0 + i, sl] = acc[inc0 + i, sl] + acc[tmp0 + i, sl]
            return c2

        def zero_inc(i, c2):
            for k in range(8):
                acc[inc0 + i, pl.ds(k * 16, 16)] = zero16
            return c2

        soff = pl.multiple_of(s * sub, 8)

        for g in range(NG):
            short = g == NG - 1
            for e in range(EG):
                plsc.subcore_barrier()

                @pl.when((ng == g) & (eg == e))
                def _():
                    pltpu.sync_copy(acc.at[pl.ds(0, qp)], shared.at[0])

                plsc.subcore_barrier()
                if e == 0:
                    lax.fori_loop(0, sub, zero_inc, 0)

                def reduce_step(sz):
                    pltpu.sync_copy(shared.at[0, pl.ds(soff, sz)],
                                    acc.at[pl.ds(tmp0, sz)])
                    lax.fori_loop(0, sz, add_chunk, 0)

                if not short:
                    @pl.when(s < nred)
                    def _():
                        reduce_step(sub)
                else:
                    @pl.when(s < full_last)
                    def _():
                        reduce_step(sub)

                    if rem > 0:
                        @pl.when(s == full_last)
                        def _():
                            reduce_step(rem)

            def write_step(sz):
                pltpu.sync_copy(
                    acc.at[pl.ds(inc0, sz)],
                    out_hbm.at[c, pl.ds(g * qp + soff, sz)])

            if not short:
                @pl.when(s < nred)
                def _():
                    write_step(sub)
            else:
                @pl.when(s < full_last)
                def _():
                    write_step(sub)

                if rem > 0:
                    @pl.when(s == full_last)
                    def _():
                        write_step(rem)

    return body, acc_r, npacked, qp, np_out


def _sum_values(e2_v, i, iota16, one_lane):
    return [(e2_v[i, pl.ds(0, 16)], iota16),
            (e2_v[i, pl.ds(16, 16)], iota16 + 16)]


def _cnt_values(e2_v, i, iota16, one_lane):
    return [(one_lane, iota16)]


(_sums_body, _SUMS_R, _SUMS_NP, _SUMS_QP,
 _SUMS_NPO) = _scatter_like(EDGE_D, _sum_values)
(_cnt_body, _CNT_R, _CNT_NP, _CNT_QP,
 _CNT_NPO) = _scatter_like(16, _cnt_values)


@functools.partial(
    pl.kernel,
    out_type=jax.ShapeDtypeStruct((NC, _SUMS_NPO, 128), jnp.float32),
    mesh=_mesh(),
    compiler_params=pltpu.CompilerParams(needs_layout_passes=False),
    scratch_types=[
        pltpu.VMEM((SCH, EDGE_D), jnp.float32),
        pltpu.VMEM((SCH,), jnp.int32),
        pltpu.VMEM((_SUMS_R, 128), jnp.float32),
        pltpu.MemorySpace.VMEM_SHARED((1, _SUMS_QP, 128), jnp.float32),
    ],
)
def _scatter_sums(e2_hbm, col_hbm, outs_hbm, e2_v, idx_v, acc, shared):
    _sums_body((e2_hbm, col_hbm, outs_hbm), e2_v, idx_v, acc, shared)


@functools.partial(
    pl.kernel,
    out_type=jax.ShapeDtypeStruct((NC, _CNT_NPO, 128), jnp.float32),
    mesh=_mesh(),
    compiler_params=pltpu.CompilerParams(needs_layout_passes=False),
    scratch_types=[
        pltpu.VMEM((SCH,), jnp.int32),
        pltpu.VMEM((_CNT_R, 128), jnp.float32),
        pltpu.MemorySpace.VMEM_SHARED((1, _CNT_QP, 128), jnp.float32),
    ],
)
def _count_edges(col_hbm, outc_hbm, idx_v, acc, shared):
    _cnt_body((None, col_hbm, outc_hbm), None, idx_v, acc, shared)


def _encoder_body(x_ref, act_ref, wx1, wx2, bx, a1, b1, be1,
                  h_ref, t_ref):
    h = x_ref[...] @ wx1[...] + act_ref[...] @ wx2[...] + bx[...]
    h = jnp.maximum(h, 0.0)
    h_ref[...] = h
    t_ref[...] = jnp.concatenate(
        [h @ a1[...] + be1[...], h @ b1[...]], axis=-1)


def _edge1_body(attr_ref, s_ref, we, be, c1, w2, b2, e2_ref):
    ea = jnp.maximum(attr_ref[...] @ we[...] + be[...], 0.0)
    hid = jnp.maximum(s_ref[...] + ea @ c1[...], 0.0)
    e2_ref[...] = ea + hid @ w2[...] + b2[...]


def _edge2_body(e2p_ref, s_ref, g, b, c2, w2, b2, e2_ref):
    e2p = e2p_ref[...]
    m = jnp.mean(e2p, axis=-1, keepdims=True)
    v = jnp.mean((e2p - m) ** 2, axis=-1, keepdims=True)
    ea = (e2p - m) * lax.rsqrt(v + 1e-5) * g[...] + b[...]
    hid = jnp.maximum(s_ref[...] + ea @ c2[...], 0.0)
    e2_ref[...] = ea + hid @ w2[...] + b2[...]


def _node_update(h, sums, cnt, wn1h, wn1a, bn1, wn2, bn2, gx, bxn):
    agg = sums / jnp.maximum(cnt, 1.0)
    z = jnp.maximum(h @ wn1h[...] + agg @ wn1a[...] + bn1[...], 0.0)
    x2 = h + z @ wn2[...] + bn2[...]
    m = jnp.mean(x2, axis=-1, keepdims=True)
    v = jnp.mean((x2 - m) ** 2, axis=-1, keepdims=True)
    return (x2 - m) * lax.rsqrt(v + 1e-5) * gx[...] + bxn[...]


def _node1_body(h_ref, ps_ref, pc_ref, wn1h, wn1a, bn1, wn2, bn2, gx, bxn,
                a2, b2w, be12, hn_ref, t_ref):
    sums = ps_ref[0] + ps_ref[1]
    cnt = pc_ref[0, :, 0:1] + pc_ref[1, :, 0:1]
    hn = _node_update(h_ref[...], sums, cnt, wn1h, wn1a, bn1,
                      wn2, bn2, gx, bxn)
    hn_ref[...] = hn
    t_ref[...] = jnp.concatenate(
        [hn @ a2[...] + be12[...], hn @ b2w[...]], axis=-1)


def _node2_body(h_ref, ps_ref, pc_ref, wn1h, wn1a, bn1, wn2, bn2, gx, bxn,
                woutt, bout, batch_ref, out_ref):
    sums = ps_ref[0] + ps_ref[1]
    cnt = pc_ref[0, :, 0:1] + pc_ref[1, :, 0:1]
    hn = _node_update(h_ref[...], sums, cnt, wn1h, wn1a, bn1,
                      wn2, bn2, gx, bxn)
    v = jnp.sum(hn * woutt[...], axis=-1, keepdims=True) + bout[...]
    gid = lax.broadcasted_iota(jnp.int32, (1, NUM_GRAPHS), 1)
    onehot = (batch_ref[...] == gid).astype(jnp.float32)
    gsum = jnp.sum(onehot * v, axis=0)
    gcnt = jnp.sum(onehot, axis=0)
    out_ref[...] = (gsum / jnp.maximum(gcnt, 1.0))[:, None]


def _full(shape, dtype=jnp.float32):
    return jax.ShapeDtypeStruct(shape, dtype)


def _encoder(x, action, wx1, wx2, bx, a1, b1, be1):
    return pl.pallas_call(
        _encoder_body,
        out_shape=(_full((N, NODE_D)), _full((N, 2 * NODE_D))),
    )(x, action, wx1, wx2, bx, a1, b1, be1)


def _make_edge_call(body, first_width, *ws):
    in_specs = [
        pl.BlockSpec((TE, first_width), lambda i: (i, 0)),
        pl.BlockSpec((TE, NODE_D), lambda i: (i, 0)),
    ] + [pl.BlockSpec(w.shape, lambda i: (0, 0)) for w in ws]
    return pl.pallas_call(
        body,
        grid=(GRID_E,),
        in_specs=in_specs,
        out_specs=pl.BlockSpec((TE, EDGE_D), lambda i: (i, 0)),
        out_shape=_full((E, EDGE_D)),
        compiler_params=pltpu.CompilerParams(
            dimension_semantics=("arbitrary",)),
    )


def kernel(x, edge_index, edge_attr, batch, action, params):
    row = edge_index[0].astype(jnp.int32)
    col = edge_index[1].astype(jnp.int32)
    blk1, blk2 = params["blocks"][0], params["blocks"][1]

    Wx = params["Wx"]
    wx1, wx2 = Wx[: x.shape[1]], Wx[x.shape[1]:]
    bx = params["bx"].reshape(1, NODE_D)
    we = params["We"]
    be = params["be"].reshape(1, EDGE_D)

    def esplit(blk):
        W = blk["We1"]
        return (W[:NODE_D], W[NODE_D:2 * NODE_D], W[2 * NODE_D:],
                blk["be1"].reshape(1, -1))

    a1, b1w, c1, be11 = esplit(blk1)
    a2, b2w, c2, be12 = esplit(blk2)

    def nsplit(blk):
        W = blk["Wn1"]
        return (W[:NODE_D], W[NODE_D:], blk["bn1"].reshape(1, -1),
                blk["Wn2"], blk["bn2"].reshape(1, -1),
                blk["gx"].reshape(1, -1), blk["bxn"].reshape(1, -1))

    n1 = nsplit(blk1)
    n2 = nsplit(blk2)

    h, t1 = _encoder(x, action, wx1, wx2, bx, a1, b1w, be11)

    s1 = _gather_sum(t1, row, col)
    ew1 = (we, be, c1, blk1["We2"], blk1["be2"].reshape(1, -1))
    e2_1 = _make_edge_call(_edge1_body, edge_attr.shape[1],
                           *ew1)(edge_attr, s1, *ew1)
    pc = _count_edges(col)[:, :_CNT_NP].reshape(NC, N, 16)
    ps1 = _scatter_sums(e2_1, col)[:, :_SUMS_NP].reshape(NC, N, EDGE_D)

    h2, t2 = pl.pallas_call(
        _node1_body,
        out_shape=(_full((N, NODE_D)), _full((N, 2 * NODE_D))),
    )(h, ps1, pc, *n1, a2, b2w, be12)

    s2 = _gather_sum(t2, row, col)
    ew2 = (blk1["ge"].reshape(1, -1), blk1["ben"].reshape(1, -1),
           c2, blk2["We2"], blk2["be2"].reshape(1, -1))
    e2_2 = _make_edge_call(_edge2_body, EDGE_D,
                           *ew2)(e2_1, s2, *ew2)
    ps2 = _scatter_sums(e2_2, col)[:, :_SUMS_NP].reshape(NC, N, EDGE_D)

    woutt = params["Wout"].reshape(1, NODE_D)
    bout = params["bout"].reshape(1, 1)
    batch2d = batch.astype(jnp.int32).reshape(N, 1)

    out = pl.pallas_call(
        _node2_body,
        out_shape=_full((NUM_GRAPHS, 1)),
    )(h2, ps2, pc, *n2, woutt, bout, batch2d)
    return out


# speedup vs baseline: 2.2649x; 1.1634x over previous
import functools

import jax
import jax.numpy as jnp
from jax import lax
from jax.experimental import pallas as pl
from jax.experimental.pallas import tpu as pltpu
from jax.experimental.pallas import tpu_sc as plsc

N = 10000
E = 320000
NODE_D = 64
EDGE_D = 32
NUM_GRAPHS = 64

NC = 2
NS = 16
NW = NC * NS
EPW = E // NW
CHUNK = 80
NCHUNK = EPW // CHUNK

EG = 4
NG = 4
EPG = E // (NC * EG)
SCH = 200
SNCH = EPG // SCH
QS = 2560

TE = 8000
GRID_E = E // TE


def _mesh():
    return plsc.VectorSubcoreMesh(core_axis_name="c", subcore_axis_name="s")


@functools.partial(
    pl.kernel,
    out_type=jax.ShapeDtypeStruct((E, NODE_D), jnp.float32),
    mesh=_mesh(),
    scratch_types=[
        [pltpu.VMEM((CHUNK,), jnp.int32) for _ in range(2)],
        [pltpu.VMEM((CHUNK,), jnp.int32) for _ in range(2)],
        [pltpu.VMEM((CHUNK, 2 * NODE_D), jnp.float32) for _ in range(2)],
        [pltpu.VMEM((CHUNK, 2 * NODE_D), jnp.float32) for _ in range(2)],
        [pltpu.VMEM((CHUNK, NODE_D), jnp.float32) for _ in range(2)],
        [pltpu.SemaphoreType.DMA for _ in range(2)],
        [pltpu.SemaphoreType.DMA for _ in range(2)],
        [pltpu.SemaphoreType.DMA for _ in range(2)],
    ],
)
def _gather_sum(t_hbm, row_hbm, col_hbm, out_hbm,
                idx_r, idx_c, bufr, bufc, bufs, gsem, isem, osem):
    wid = lax.axis_index("s") * NC + lax.axis_index("c")
    base0 = wid * EPW

    def start(j, p):
        base = base0 + j * CHUNK
        pltpu.async_copy(row_hbm.at[pl.ds(base, CHUNK)], idx_r[p],
                         isem[p]).wait()
        pltpu.async_copy(col_hbm.at[pl.ds(base, CHUNK)], idx_c[p],
                         isem[p]).wait()
        pltpu.async_copy(t_hbm.at[idx_r[p]], bufr[p], gsem[p])
        pltpu.async_copy(t_hbm.at[idx_c[p]], bufc[p], gsem[p])

    def finish(j, p):
        base = base0 + j * CHUNK
        pltpu.make_async_copy(t_hbm.at[idx_r[p]], bufr[p], gsem[p]).wait()
        pltpu.make_async_copy(t_hbm.at[idx_c[p]], bufc[p], gsem[p]).wait()

        def add_row(i, c2):
            for k in range(NODE_D // 16):
                sl = pl.ds(k * 16, 16)
                sl_q = pl.ds(NODE_D + k * 16, 16)
                bufs[p][i, sl] = bufr[p][i, sl] + bufc[p][i, sl_q]
            return c2

        lax.fori_loop(0, CHUNK, add_row, 0)
        pltpu.async_copy(bufs[p], out_hbm.at[pl.ds(base, CHUNK)], osem[p])

    start(0, 0)

    def chunk_pair(jj, carry):
        j = jj * 2

        @pl.when(jj > 0)
        def _():
            pltpu.make_async_copy(bufs[0], out_hbm.at[pl.ds(base0, CHUNK)],
                                  osem[0]).wait()

        start(j + 1, 1)
        finish(j, 0)

        @pl.when(jj > 0)
        def _():
            pltpu.make_async_copy(bufs[1], out_hbm.at[pl.ds(base0, CHUNK)],
                                  osem[1]).wait()

        @pl.when(jj + 1 < NCHUNK // 2)
        def _():
            start(j + 2, 0)

        finish(j + 1, 1)
        return carry

    lax.fori_loop(0, NCHUNK // 2, chunk_pair, 0)
    if NCHUNK % 2 == 1:
        pltpu.make_async_copy(bufs[0], out_hbm.at[pl.ds(base0, CHUNK)],
                              osem[0]).wait()
        start(NCHUNK - 1, 0)
        finish(NCHUNK - 1, 0)
    pltpu.make_async_copy(bufs[0], out_hbm.at[pl.ds(base0, CHUNK)],
                          osem[0]).wait()
    pltpu.make_async_copy(bufs[1], out_hbm.at[pl.ds(base0, CHUNK)],
                          osem[1]).wait()


def _scatter_like(width, values_fn):
    pack = 128 // width
    qp = QS // pack
    qp_last = (N - (NG - 1) * QS) // pack
    sub = 40
    nred = qp // sub
    dummy_row = qp
    tmp0 = qp + 8
    inc0 = tmp0 + sub
    acc_r = inc0 + sub
    shift = pack.bit_length() - 1
    npacked = N // pack

    full_last = qp_last // sub
    rem = -(-(qp_last - full_last * sub) // 8) * 8
    np_out = (NG - 1) * qp + full_last * sub + rem

    def body(refs, e2_v, idx_v, acc, shared):
        c = lax.axis_index("c")
        s = lax.axis_index("s")
        eg = s // NG
        ng = s % NG
        lo = ng * QS
        hi = jnp.minimum(lo + QS, N)
        zero16 = jnp.zeros((16,), jnp.float32)
        iota16 = lax.iota(jnp.int32, 16)
        zero_i16 = iota16 * 0
        one_lane = (1 - jnp.minimum(iota16, 1)).astype(jnp.float32)
        e2_hbm, col_hbm, out_hbm = refs

        def fill_zero(i, carry):
            for k in range(8):
                acc[i, pl.ds(k * 16, 16)] = zero16
            return carry

        lax.fori_loop(0, acc_r, fill_zero, 0)

        base_e = (c * EG + eg) * EPG

        def chunk_body(j, carry):
            base = base_e + j * SCH
            pltpu.sync_copy(col_hbm.at[pl.ds(base, SCH)], idx_v)
            if e2_hbm is not None:
                pltpu.sync_copy(e2_hbm.at[pl.ds(base, SCH)], e2_v)

            def group_body(g, c2):
                cids = idx_v[pl.ds(g * 16, 16)]
                q = cids - lo
                ok = (cids >= lo) & (cids < hi)
                rows = jnp.where(ok, lax.shift_right_logical(q, shift),
                                 dummy_row)
                offs = jnp.where(ok, (q & (pack - 1)) * width, 0)
                for l in range(16):
                    i = g * 16 + l
                    r16 = zero_i16 + rows[l]
                    for val, lidx in values_fn(e2_v, i, iota16, one_lane):
                        plsc.addupdate_scatter(acc, [r16, lidx + offs[l]],
                                               val)
                return c2

            lax.fori_loop(0, SCH // 16, group_body, 0)
            return carry

        lax.fori_loop(0, SNCH, chunk_body, 0)

        def add_chunk(i, c2):
            for k in range(8):
                sl = pl.ds(k * 16, 16)
                acc[inc0 + i, sl] = acc[inc0 + i, sl] + acc[tmp0 + i, sl]
            return c2

        def zero_inc(i, c2):
            for k in range(8):
                acc[inc0 + i, pl.ds(k * 16, 16)] = zero16
            return c2

        soff = pl.multiple_of(s * sub, 8)

        for g in range(NG):
            short = g == NG - 1
            for e in range(EG):
                plsc.subcore_barrier()

                @pl.when((ng == g) & (eg == e))
                def _():
                    pltpu.sync_copy(acc.at[pl.ds(0, qp)], shared.at[0])

                plsc.subcore_barrier()
                if e == 0:
                    lax.fori_loop(0, sub, zero_inc, 0)

                def reduce_step(sz):
                    pltpu.sync_copy(shared.at[0, pl.ds(soff, sz)],
                                    acc.at[pl.ds(tmp0, sz)])
                    lax.fori_loop(0, sz, add_chunk, 0)

                if not short:
                    @pl.when(s < nred)
                    def _():
                        reduce_step(sub)
                else:
                    @pl.when(s < full_last)
                    def _():
                        reduce_step(sub)

                    if rem > 0:
                        @pl.when(s == full_last)
                        def _():
                            reduce_step(rem)

            def write_step(sz):
                pltpu.sync_copy(
                    acc.at[pl.ds(inc0, sz)],
                    out_hbm.at[c, pl.ds(g * qp + soff, sz)])

            if not short:
                @pl.when(s < nred)
                def _():
                    write_step(sub)
            else:
                @pl.when(s < full_last)
                def _():
                    write_step(sub)

                if rem > 0:
                    @pl.when(s == full_last)
                    def _():
                        write_step(rem)

    return body, acc_r, npacked, qp, np_out


def _sum_values(e2_v, i, iota16, one_lane):
    return [(e2_v[i, pl.ds(0, 16)], iota16),
            (e2_v[i, pl.ds(16, 16)], iota16 + 16)]


def _cnt_values(e2_v, i, iota16, one_lane):
    return [(one_lane, iota16)]


(_sums_body, _SUMS_R, _SUMS_NP, _SUMS_QP,
 _SUMS_NPO) = _scatter_like(EDGE_D, _sum_values)
(_cnt_body, _CNT_R, _CNT_NP, _CNT_QP,
 _CNT_NPO) = _scatter_like(16, _cnt_values)


@functools.partial(
    pl.kernel,
    out_type=jax.ShapeDtypeStruct((NC, _SUMS_NPO, 128), jnp.float32),
    mesh=_mesh(),
    compiler_params=pltpu.CompilerParams(needs_layout_passes=False),
    scratch_types=[
        pltpu.VMEM((SCH, EDGE_D), jnp.float32),
        pltpu.VMEM((SCH,), jnp.int32),
        pltpu.VMEM((_SUMS_R, 128), jnp.float32),
        pltpu.MemorySpace.VMEM_SHARED((1, _SUMS_QP, 128), jnp.float32),
    ],
)
def _scatter_sums(e2_hbm, col_hbm, outs_hbm, e2_v, idx_v, acc, shared):
    _sums_body((e2_hbm, col_hbm, outs_hbm), e2_v, idx_v, acc, shared)


@functools.partial(
    pl.kernel,
    out_type=jax.ShapeDtypeStruct((NC, _CNT_NPO, 128), jnp.float32),
    mesh=_mesh(),
    compiler_params=pltpu.CompilerParams(needs_layout_passes=False),
    scratch_types=[
        pltpu.VMEM((SCH,), jnp.int32),
        pltpu.VMEM((_CNT_R, 128), jnp.float32),
        pltpu.MemorySpace.VMEM_SHARED((1, _CNT_QP, 128), jnp.float32),
    ],
)
def _count_edges(col_hbm, outc_hbm, idx_v, acc, shared):
    _cnt_body((None, col_hbm, outc_hbm), None, idx_v, acc, shared)


def _encoder_body(x_ref, act_ref, wx1, wx2, bx, a1, b1, be1,
                  h_ref, t_ref):
    h = x_ref[...] @ wx1[...] + act_ref[...] @ wx2[...] + bx[...]
    h = jnp.maximum(h, 0.0)
    h_ref[...] = h
    t_ref[...] = jnp.concatenate(
        [h @ a1[...] + be1[...], h @ b1[...]], axis=-1)


def _edge1_body(attr_ref, s_ref, we, be, c1, w2, b2, e2_ref):
    ea = jnp.maximum(attr_ref[...] @ we[...] + be[...], 0.0)
    hid = jnp.maximum(s_ref[...] + ea @ c1[...], 0.0)
    e2_ref[...] = ea + hid @ w2[...] + b2[...]


def _edge2_body(e2p_ref, s_ref, g, b, c2, w2, b2, e2_ref):
    e2p = e2p_ref[...]
    m = jnp.mean(e2p, axis=-1, keepdims=True)
    v = jnp.mean((e2p - m) ** 2, axis=-1, keepdims=True)
    ea = (e2p - m) * lax.rsqrt(v + 1e-5) * g[...] + b[...]
    hid = jnp.maximum(s_ref[...] + ea @ c2[...], 0.0)
    e2_ref[...] = ea + hid @ w2[...] + b2[...]


def _node_update(h, sums, cnt, wn1h, wn1a, bn1, wn2, bn2, gx, bxn):
    agg = sums / jnp.maximum(cnt, 1.0)
    z = jnp.maximum(h @ wn1h[...] + agg @ wn1a[...] + bn1[...], 0.0)
    x2 = h + z @ wn2[...] + bn2[...]
    m = jnp.mean(x2, axis=-1, keepdims=True)
    v = jnp.mean((x2 - m) ** 2, axis=-1, keepdims=True)
    return (x2 - m) * lax.rsqrt(v + 1e-5) * gx[...] + bxn[...]


def _node1_body(h_ref, ps_ref, pc_ref, wn1h, wn1a, bn1, wn2, bn2, gx, bxn,
                a2, b2w, be12, hn_ref, t_ref):
    sums = ps_ref[0] + ps_ref[1]
    cnt = pc_ref[0, :, 0:1] + pc_ref[1, :, 0:1]
    hn = _node_update(h_ref[...], sums, cnt, wn1h, wn1a, bn1,
                      wn2, bn2, gx, bxn)
    hn_ref[...] = hn
    t_ref[...] = jnp.concatenate(
        [hn @ a2[...] + be12[...], hn @ b2w[...]], axis=-1)


def _node2_body(h_ref, ps_ref, pc_ref, wn1h, wn1a, bn1, wn2, bn2, gx, bxn,
                woutt, bout, batch_ref, out_ref):
    sums = ps_ref[0] + ps_ref[1]
    cnt = pc_ref[0, :, 0:1] + pc_ref[1, :, 0:1]
    hn = _node_update(h_ref[...], sums, cnt, wn1h, wn1a, bn1,
                      wn2, bn2, gx, bxn)
    v = jnp.sum(hn * woutt[...], axis=-1, keepdims=True) + bout[...]
    gid = lax.broadcasted_iota(jnp.int32, (1, NUM_GRAPHS), 1)
    onehot = (batch_ref[...] == gid).astype(jnp.float32)
    gsum = jnp.sum(onehot * v, axis=0)
    gcnt = jnp.sum(onehot, axis=0)
    out_ref[...] = (gsum / jnp.maximum(gcnt, 1.0))[:, None]


def _full(shape, dtype=jnp.float32):
    return jax.ShapeDtypeStruct(shape, dtype)


def _encoder(x, action, wx1, wx2, bx, a1, b1, be1):
    return pl.pallas_call(
        _encoder_body,
        out_shape=(_full((N, NODE_D)), _full((N, 2 * NODE_D))),
    )(x, action, wx1, wx2, bx, a1, b1, be1)


def _make_edge_call(body, first_width, *ws):
    in_specs = [
        pl.BlockSpec((TE, first_width), lambda i: (i, 0)),
        pl.BlockSpec((TE, NODE_D), lambda i: (i, 0)),
    ] + [pl.BlockSpec(w.shape, lambda i: (0, 0)) for w in ws]
    return pl.pallas_call(
        body,
        grid=(GRID_E,),
        in_specs=in_specs,
        out_specs=pl.BlockSpec((TE, EDGE_D), lambda i: (i, 0)),
        out_shape=_full((E, EDGE_D)),
        compiler_params=pltpu.CompilerParams(
            dimension_semantics=("arbitrary",)),
    )


def kernel(x, edge_index, edge_attr, batch, action, params):
    row = edge_index[0].astype(jnp.int32)
    col = edge_index[1].astype(jnp.int32)
    blk1, blk2 = params["blocks"][0], params["blocks"][1]

    Wx = params["Wx"]
    wx1, wx2 = Wx[: x.shape[1]], Wx[x.shape[1]:]
    bx = params["bx"].reshape(1, NODE_D)
    we = params["We"]
    be = params["be"].reshape(1, EDGE_D)

    def esplit(blk):
        W = blk["We1"]
        return (W[:NODE_D], W[NODE_D:2 * NODE_D], W[2 * NODE_D:],
                blk["be1"].reshape(1, -1))

    a1, b1w, c1, be11 = esplit(blk1)
    a2, b2w, c2, be12 = esplit(blk2)

    def nsplit(blk):
        W = blk["Wn1"]
        return (W[:NODE_D], W[NODE_D:], blk["bn1"].reshape(1, -1),
                blk["Wn2"], blk["bn2"].reshape(1, -1),
                blk["gx"].reshape(1, -1), blk["bxn"].reshape(1, -1))

    n1 = nsplit(blk1)
    n2 = nsplit(blk2)

    h, t1 = _encoder(x, action, wx1, wx2, bx, a1, b1w, be11)

    s1 = _gather_sum(t1, row, col)
    ew1 = (we, be, c1, blk1["We2"], blk1["be2"].reshape(1, -1))
    e2_1 = _make_edge_call(_edge1_body, edge_attr.shape[1],
                           *ew1)(edge_attr, s1, *ew1)
    pc = _count_edges(col)[:, :_CNT_NP].reshape(NC, N, 16)
    ps1 = _scatter_sums(e2_1, col)[:, :_SUMS_NP].reshape(NC, N, EDGE_D)

    h2, t2 = pl.pallas_call(
        _node1_body,
        out_shape=(_full((N, NODE_D)), _full((N, 2 * NODE_D))),
    )(h, ps1, pc, *n1, a2, b2w, be12)

    s2 = _gather_sum(t2, row, col)
    ew2 = (blk1["ge"].reshape(1, -1), blk1["ben"].reshape(1, -1),
           c2, blk2["We2"], blk2["be2"].reshape(1, -1))
    e2_2 = _make_edge_call(_edge2_body, EDGE_D,
                           *ew2)(e2_1, s2, *ew2)
    ps2 = _scatter_sums(e2_2, col)[:, :_SUMS_NP].reshape(NC, N, EDGE_D)

    woutt = params["Wout"].reshape(1, NODE_D)
    bout = params["bout"].reshape(1, 1)
    batch2d = batch.astype(jnp.int32).reshape(N, 1)

    out = pl.pallas_call(
        _node2_body,
        out_shape=_full((NUM_GRAPHS, 1)),
    )(h2, ps2, pc, *n2, woutt, bout, batch2d)
    return out


# speedup vs baseline: 2.8176x; 1.2440x over previous
import functools

import jax
import jax.numpy as jnp
from jax import lax
from jax.experimental import pallas as pl
from jax.experimental.pallas import tpu as pltpu
from jax.experimental.pallas import tpu_sc as plsc

N = 10000
E = 320000
NODE_D = 64
EDGE_D = 32
NUM_GRAPHS = 64

NC = 2
NS = 16
NW = NC * NS
EPW = E // NW
CHUNK = 80
NCHUNK = EPW // CHUNK

EG = 4
NG = 4
EPG = E // (NC * EG)
SCH = 80
SNCH = EPG // SCH
QS = 2560

TE = 8000
GRID_E = E // TE


def _mesh():
    return plsc.VectorSubcoreMesh(core_axis_name="c", subcore_axis_name="s")


@functools.partial(
    pl.kernel,
    out_type=jax.ShapeDtypeStruct((E, NODE_D), jnp.float32),
    mesh=_mesh(),
    scratch_types=[
        [pltpu.VMEM((CHUNK,), jnp.int32) for _ in range(2)],
        [pltpu.VMEM((CHUNK,), jnp.int32) for _ in range(2)],
        [pltpu.VMEM((CHUNK, 2 * NODE_D), jnp.float32) for _ in range(2)],
        [pltpu.VMEM((CHUNK, 2 * NODE_D), jnp.float32) for _ in range(2)],
        [pltpu.VMEM((CHUNK, NODE_D), jnp.float32) for _ in range(2)],
        [pltpu.SemaphoreType.DMA for _ in range(2)],
        [pltpu.SemaphoreType.DMA for _ in range(2)],
        [pltpu.SemaphoreType.DMA for _ in range(2)],
    ],
)
def _gather_sum(t_hbm, row_hbm, col_hbm, out_hbm,
                idx_r, idx_c, bufr, bufc, bufs, gsem, isem, osem):
    wid = lax.axis_index("s") * NC + lax.axis_index("c")
    base0 = wid * EPW

    def start(j, p):
        base = base0 + j * CHUNK
        pltpu.async_copy(row_hbm.at[pl.ds(base, CHUNK)], idx_r[p],
                         isem[p]).wait()
        pltpu.async_copy(col_hbm.at[pl.ds(base, CHUNK)], idx_c[p],
                         isem[p]).wait()
        pltpu.async_copy(t_hbm.at[idx_r[p]], bufr[p], gsem[p])
        pltpu.async_copy(t_hbm.at[idx_c[p]], bufc[p], gsem[p])

    def finish(j, p):
        base = base0 + j * CHUNK
        pltpu.make_async_copy(t_hbm.at[idx_r[p]], bufr[p], gsem[p]).wait()
        pltpu.make_async_copy(t_hbm.at[idx_c[p]], bufc[p], gsem[p]).wait()

        def add_row(i, c2):
            for k in range(NODE_D // 16):
                sl = pl.ds(k * 16, 16)
                sl_q = pl.ds(NODE_D + k * 16, 16)
                bufs[p][i, sl] = bufr[p][i, sl] + bufc[p][i, sl_q]
            return c2

        lax.fori_loop(0, CHUNK, add_row, 0)
        pltpu.async_copy(bufs[p], out_hbm.at[pl.ds(base, CHUNK)], osem[p])

    start(0, 0)

    def chunk_pair(jj, carry):
        j = jj * 2

        @pl.when(jj > 0)
        def _():
            pltpu.make_async_copy(bufs[0], out_hbm.at[pl.ds(base0, CHUNK)],
                                  osem[0]).wait()

        start(j + 1, 1)
        finish(j, 0)

        @pl.when(jj > 0)
        def _():
            pltpu.make_async_copy(bufs[1], out_hbm.at[pl.ds(base0, CHUNK)],
                                  osem[1]).wait()

        @pl.when(jj + 1 < NCHUNK // 2)
        def _():
            start(j + 2, 0)

        finish(j + 1, 1)
        return carry

    lax.fori_loop(0, NCHUNK // 2, chunk_pair, 0)
    if NCHUNK % 2 == 1:
        pltpu.make_async_copy(bufs[0], out_hbm.at[pl.ds(base0, CHUNK)],
                              osem[0]).wait()
        start(NCHUNK - 1, 0)
        finish(NCHUNK - 1, 0)
    pltpu.make_async_copy(bufs[0], out_hbm.at[pl.ds(base0, CHUNK)],
                          osem[0]).wait()
    pltpu.make_async_copy(bufs[1], out_hbm.at[pl.ds(base0, CHUNK)],
                          osem[1]).wait()


def _scatter_like(width, values_fn):
    pack = 128 // width
    qp = QS // pack
    qp_last = (N - (NG - 1) * QS) // pack
    sub = 40
    nred = qp // sub
    dummy_row = qp
    tmp0 = qp + 8
    inc0 = tmp0 + sub
    acc_r = inc0 + sub
    shift = pack.bit_length() - 1
    npacked = N // pack

    full_last = qp_last // sub
    rem = -(-(qp_last - full_last * sub) // 8) * 8
    np_out = (NG - 1) * qp + full_last * sub + rem

    def body(refs, e2_v, idx_v, acc, shared, isem, esem):
        c = lax.axis_index("c")
        s = lax.axis_index("s")
        eg = s // NG
        ng = s % NG
        lo = ng * QS
        hi = jnp.minimum(lo + QS, N)
        zero16 = jnp.zeros((16,), jnp.float32)
        iota16 = lax.iota(jnp.int32, 16)
        zero_i16 = iota16 * 0
        one_lane = (1 - jnp.minimum(iota16, 1)).astype(jnp.float32)
        e2_hbm, col_hbm, out_hbm = refs

        def fill_zero(i, carry):
            for k in range(8):
                acc[i, pl.ds(k * 16, 16)] = zero16
            return carry

        lax.fori_loop(0, acc_r, fill_zero, 0)

        base_e = (c * EG + eg) * EPG

        def stage(j, p):
            base = base_e + j * SCH
            pltpu.async_copy(col_hbm.at[pl.ds(base, SCH)], idx_v[p],
                             isem[p])
            if e2_hbm is not None:
                pltpu.async_copy(e2_hbm.at[pl.ds(base, SCH)], e2_v[p],
                                 esem[p])

        def consume(j, p):
            base = base_e + j * SCH
            pltpu.make_async_copy(col_hbm.at[pl.ds(base, SCH)], idx_v[p],
                                  isem[p]).wait()
            if e2_hbm is not None:
                pltpu.make_async_copy(e2_hbm.at[pl.ds(base, SCH)], e2_v[p],
                                      esem[p]).wait()

            def group_body(g, c2):
                cids = idx_v[p][pl.ds(g * 16, 16)]
                q = cids - lo
                ok = (cids >= lo) & (cids < hi)
                rows = jnp.where(ok, lax.shift_right_logical(q, shift),
                                 dummy_row)
                offs = jnp.where(ok, (q & (pack - 1)) * width, 0)
                for l in range(16):
                    i = g * 16 + l
                    r16 = zero_i16 + rows[l]
                    for val, lidx in values_fn(
                            e2_v[p] if e2_v is not None else None,
                            i, iota16, one_lane):
                        plsc.addupdate_scatter(acc, [r16, lidx + offs[l]],
                                               val)
                return c2

            lax.fori_loop(0, SCH // 16, group_body, 0)

        stage(0, 0)

        def chunk_pair(jj, carry):
            j = jj * 2
            stage(j + 1, 1)
            consume(j, 0)

            @pl.when(jj + 1 < SNCH // 2)
            def _():
                stage(j + 2, 0)

            consume(j + 1, 1)
            return carry

        lax.fori_loop(0, SNCH // 2, chunk_pair, 0)

        def add_chunk(i, c2):
            for k in range(8):
                sl = pl.ds(k * 16, 16)
                acc[inc0 + i, sl] = acc[inc0 + i, sl] + acc[tmp0 + i, sl]
            return c2

        def zero_inc(i, c2):
            for k in range(8):
                acc[inc0 + i, pl.ds(k * 16, 16)] = zero16
            return c2

        soff = pl.multiple_of(s * sub, 8)

        for g in range(NG):
            short = g == NG - 1
            for e in range(EG):
                plsc.subcore_barrier()

                @pl.when((ng == g) & (eg == e))
                def _():
                    pltpu.sync_copy(acc.at[pl.ds(0, qp)], shared.at[0])

                plsc.subcore_barrier()
                if e == 0:
                    lax.fori_loop(0, sub, zero_inc, 0)

                def reduce_step(sz):
                    pltpu.sync_copy(shared.at[0, pl.ds(soff, sz)],
                                    acc.at[pl.ds(tmp0, sz)])
                    lax.fori_loop(0, sz, add_chunk, 0)

                if not short:
                    @pl.when(s < nred)
                    def _():
                        reduce_step(sub)
                else:
                    @pl.when(s < full_last)
                    def _():
                        reduce_step(sub)

                    if rem > 0:
                        @pl.when(s == full_last)
                        def _():
                            reduce_step(rem)

            def write_step(sz):
                pltpu.sync_copy(
                    acc.at[pl.ds(inc0, sz)],
                    out_hbm.at[c, pl.ds(g * qp + soff, sz)])

            if not short:
                @pl.when(s < nred)
                def _():
                    write_step(sub)
            else:
                @pl.when(s < full_last)
                def _():
                    write_step(sub)

                if rem > 0:
                    @pl.when(s == full_last)
                    def _():
                        write_step(rem)

    return body, acc_r, npacked, qp, np_out


def _sum_values(e2_v, i, iota16, one_lane):
    return [(e2_v[i, pl.ds(0, 16)], iota16),
            (e2_v[i, pl.ds(16, 16)], iota16 + 16)]


def _cnt_values(e2_v, i, iota16, one_lane):
    return [(one_lane, iota16)]


(_sums_body, _SUMS_R, _SUMS_NP, _SUMS_QP,
 _SUMS_NPO) = _scatter_like(EDGE_D, _sum_values)
(_cnt_body, _CNT_R, _CNT_NP, _CNT_QP,
 _CNT_NPO) = _scatter_like(16, _cnt_values)


@functools.partial(
    pl.kernel,
    out_type=jax.ShapeDtypeStruct((NC, _SUMS_NPO, 128), jnp.float32),
    mesh=_mesh(),
    compiler_params=pltpu.CompilerParams(needs_layout_passes=False),
    scratch_types=[
        [pltpu.VMEM((SCH, EDGE_D), jnp.float32) for _ in range(2)],
        [pltpu.VMEM((SCH,), jnp.int32) for _ in range(2)],
        pltpu.VMEM((_SUMS_R, 128), jnp.float32),
        pltpu.MemorySpace.VMEM_SHARED((1, _SUMS_QP, 128), jnp.float32),
        [pltpu.SemaphoreType.DMA for _ in range(2)],
        [pltpu.SemaphoreType.DMA for _ in range(2)],
    ],
)
def _scatter_sums(e2_hbm, col_hbm, outs_hbm, e2_v, idx_v, acc, shared,
                  isem, esem):
    _sums_body((e2_hbm, col_hbm, outs_hbm), e2_v, idx_v, acc, shared,
               isem, esem)


@functools.partial(
    pl.kernel,
    out_type=jax.ShapeDtypeStruct((NC, _CNT_NPO, 128), jnp.float32),
    mesh=_mesh(),
    compiler_params=pltpu.CompilerParams(needs_layout_passes=False),
    scratch_types=[
        [pltpu.VMEM((SCH,), jnp.int32) for _ in range(2)],
        pltpu.VMEM((_CNT_R, 128), jnp.float32),
        pltpu.MemorySpace.VMEM_SHARED((1, _CNT_QP, 128), jnp.float32),
        [pltpu.SemaphoreType.DMA for _ in range(2)],
    ],
)
def _count_edges(col_hbm, outc_hbm, idx_v, acc, shared, isem):
    _cnt_body((None, col_hbm, outc_hbm), None, idx_v, acc, shared,
              isem, None)


def _encoder_body(x_ref, act_ref, wx1, wx2, bx, a1, b1, be1,
                  h_ref, t_ref):
    h = x_ref[...] @ wx1[...] + act_ref[...] @ wx2[...] + bx[...]
    h = jnp.maximum(h, 0.0)
    h_ref[...] = h
    t_ref[...] = jnp.concatenate(
        [h @ a1[...] + be1[...], h @ b1[...]], axis=-1)


def _edge1_body(attr_ref, s_ref, we, be, c1, w2, b2, e2_ref):
    ea = jnp.maximum(attr_ref[...] @ we[...] + be[...], 0.0)
    hid = jnp.maximum(s_ref[...] + ea @ c1[...], 0.0)
    e2_ref[...] = ea + hid @ w2[...] + b2[...]


def _edge2_body(e2p_ref, s_ref, g, b, c2, w2, b2, e2_ref):
    e2p = e2p_ref[...]
    m = jnp.mean(e2p, axis=-1, keepdims=True)
    v = jnp.mean((e2p - m) ** 2, axis=-1, keepdims=True)
    ea = (e2p - m) * lax.rsqrt(v + 1e-5) * g[...] + b[...]
    hid = jnp.maximum(s_ref[...] + ea @ c2[...], 0.0)
    e2_ref[...] = ea + hid @ w2[...] + b2[...]


def _node_update(h, sums, cnt, wn1h, wn1a, bn1, wn2, bn2, gx, bxn):
    agg = sums / jnp.maximum(cnt, 1.0)
    z = jnp.maximum(h @ wn1h[...] + agg @ wn1a[...] + bn1[...], 0.0)
    x2 = h + z @ wn2[...] + bn2[...]
    m = jnp.mean(x2, axis=-1, keepdims=True)
    v = jnp.mean((x2 - m) ** 2, axis=-1, keepdims=True)
    return (x2 - m) * lax.rsqrt(v + 1e-5) * gx[...] + bxn[...]


def _node1_body(h_ref, ps_ref, pc_ref, wn1h, wn1a, bn1, wn2, bn2, gx, bxn,
                a2, b2w, be12, hn_ref, t_ref):
    sums = ps_ref[0] + ps_ref[1]
    cnt = pc_ref[0, :, 0:1] + pc_ref[1, :, 0:1]
    hn = _node_update(h_ref[...], sums, cnt, wn1h, wn1a, bn1,
                      wn2, bn2, gx, bxn)
    hn_ref[...] = hn
    t_ref[...] = jnp.concatenate(
        [hn @ a2[...] + be12[...], hn @ b2w[...]], axis=-1)


def _node2_body(h_ref, ps_ref, pc_ref, wn1h, wn1a, bn1, wn2, bn2, gx, bxn,
                woutt, bout, batch_ref, out_ref):
    sums = ps_ref[0] + ps_ref[1]
    cnt = pc_ref[0, :, 0:1] + pc_ref[1, :, 0:1]
    hn = _node_update(h_ref[...], sums, cnt, wn1h, wn1a, bn1,
                      wn2, bn2, gx, bxn)
    v = jnp.sum(hn * woutt[...], axis=-1, keepdims=True) + bout[...]
    gid = lax.broadcasted_iota(jnp.int32, (1, NUM_GRAPHS), 1)
    onehot = (batch_ref[...] == gid).astype(jnp.float32)
    gsum = jnp.sum(onehot * v, axis=0)
    gcnt = jnp.sum(onehot, axis=0)
    out_ref[...] = (gsum / jnp.maximum(gcnt, 1.0))[:, None]


def _full(shape, dtype=jnp.float32):
    return jax.ShapeDtypeStruct(shape, dtype)


def _encoder(x, action, wx1, wx2, bx, a1, b1, be1):
    return pl.pallas_call(
        _encoder_body,
        out_shape=(_full((N, NODE_D)), _full((N, 2 * NODE_D))),
    )(x, action, wx1, wx2, bx, a1, b1, be1)


def _make_edge_call(body, first_width, *ws):
    in_specs = [
        pl.BlockSpec((TE, first_width), lambda i: (i, 0)),
        pl.BlockSpec((TE, NODE_D), lambda i: (i, 0)),
    ] + [pl.BlockSpec(w.shape, lambda i: (0, 0)) for w in ws]
    return pl.pallas_call(
        body,
        grid=(GRID_E,),
        in_specs=in_specs,
        out_specs=pl.BlockSpec((TE, EDGE_D), lambda i: (i, 0)),
        out_shape=_full((E, EDGE_D)),
        compiler_params=pltpu.CompilerParams(
            dimension_semantics=("arbitrary",)),
    )


def kernel(x, edge_index, edge_attr, batch, action, params):
    row = edge_index[0].astype(jnp.int32)
    col = edge_index[1].astype(jnp.int32)
    blk1, blk2 = params["blocks"][0], params["blocks"][1]

    Wx = params["Wx"]
    wx1, wx2 = Wx[: x.shape[1]], Wx[x.shape[1]:]
    bx = params["bx"].reshape(1, NODE_D)
    we = params["We"]
    be = params["be"].reshape(1, EDGE_D)

    def esplit(blk):
        W = blk["We1"]
        return (W[:NODE_D], W[NODE_D:2 * NODE_D], W[2 * NODE_D:],
                blk["be1"].reshape(1, -1))

    a1, b1w, c1, be11 = esplit(blk1)
    a2, b2w, c2, be12 = esplit(blk2)

    def nsplit(blk):
        W = blk["Wn1"]
        return (W[:NODE_D], W[NODE_D:], blk["bn1"].reshape(1, -1),
                blk["Wn2"], blk["bn2"].reshape(1, -1),
                blk["gx"].reshape(1, -1), blk["bxn"].reshape(1, -1))

    n1 = nsplit(blk1)
    n2 = nsplit(blk2)

    h, t1 = _encoder(x, action, wx1, wx2, bx, a1, b1w, be11)

    s1 = _gather_sum(t1, row, col)
    ew1 = (we, be, c1, blk1["We2"], blk1["be2"].reshape(1, -1))
    e2_1 = _make_edge_call(_edge1_body, edge_attr.shape[1],
                           *ew1)(edge_attr, s1, *ew1)
    pc = _count_edges(col)[:, :_CNT_NP].reshape(NC, N, 16)
    ps1 = _scatter_sums(e2_1, col)[:, :_SUMS_NP].reshape(NC, N, EDGE_D)

    h2, t2 = pl.pallas_call(
        _node1_body,
        out_shape=(_full((N, NODE_D)), _full((N, 2 * NODE_D))),
    )(h, ps1, pc, *n1, a2, b2w, be12)

    s2 = _gather_sum(t2, row, col)
    ew2 = (blk1["ge"].reshape(1, -1), blk1["ben"].reshape(1, -1),
           c2, blk2["We2"], blk2["be2"].reshape(1, -1))
    e2_2 = _make_edge_call(_edge2_body, EDGE_D,
                           *ew2)(e2_1, s2, *ew2)
    ps2 = _scatter_sums(e2_2, col)[:, :_SUMS_NP].reshape(NC, N, EDGE_D)

    woutt = params["Wout"].reshape(1, NODE_D)
    bout = params["bout"].reshape(1, 1)
    batch2d = batch.astype(jnp.int32).reshape(N, 1)

    out = pl.pallas_call(
        _node2_body,
        out_shape=_full((NUM_GRAPHS, 1)),
    )(h2, ps2, pc, *n2, woutt, bout, batch2d)
    return out
